# baseline (speedup 1.0000x reference)
import math

import jax
import jax.numpy as jnp
import numpy as np
from jax.experimental import pallas as pl
from jax.experimental.shard_map import shard_map
from jax.sharding import Mesh, PartitionSpec as P

_EPS = 1e-12
_TWO_PI = 2.0 * math.pi


def _sincos_kernel(tod_ref, sin_ref, cos_ref):
    rad = tod_ref[...] * _TWO_PI
    sin_ref[...] = jnp.sin(rad)
    cos_ref[...] = jnp.cos(rad)


def _fused_kernel(hid_ref, pos_ref, sin_ref, cos_ref, day_ref, wcat_ref,
                  gamma_ref, beta_ref, out_ref):
    x = hid_ref[0]
    p = pos_ref[...]
    sin_t = sin_ref[0, 0]
    cos_t = cos_ref[0, 0]
    day = day_ref[0, 0]

    bs = x.shape[0]
    col = jax.lax.broadcasted_iota(jnp.int32, (bs, 16), 1)
    hot = ((col == day + 2) | (col == 9)).astype(jnp.float32)
    m = jnp.where(col == 0, sin_t, jnp.where(col == 1, cos_t, hot))
    extra = jnp.dot(m, wcat_ref[...],
                    preferred_element_type=jnp.float32)

    h = x + p + extra
    inv_h = 1.0 / h.shape[1]
    mean = jnp.sum(h, axis=1, keepdims=True) * inv_h
    msq = jnp.sum(h * h, axis=1, keepdims=True) * inv_h
    var = msq - mean * mean
    rs = jax.lax.rsqrt(var + _EPS)
    out_ref[0] = (h - mean) * rs * gamma_ref[...] + beta_ref[...]


def _local_encode(hidden_states, time_of_day, day_of_week, pos_emb, wcat,
                  gamma2, beta2):
    B, S, H = hidden_states.shape
    BS = 512
    NSB = S // BS

    sin_bs, cos_bs = pl.pallas_call(
        _sincos_kernel,
        out_shape=(jax.ShapeDtypeStruct((B, S), jnp.float32),
                   jax.ShapeDtypeStruct((B, S), jnp.float32)),
    )(time_of_day)

    sin4 = sin_bs.reshape(B, NSB, BS, 1)
    cos4 = cos_bs.reshape(B, NSB, BS, 1)
    day4 = day_of_week.astype(jnp.int32).reshape(B, NSB, BS, 1)

    grid = (NSB, B)
    return pl.pallas_call(
        _fused_kernel,
        grid=grid,
        in_specs=[
            pl.BlockSpec((1, BS, H), lambda s, b: (b, s, 0)),
            pl.BlockSpec((BS, H), lambda s, b: (s, 0)),
            pl.BlockSpec((1, 1, BS, 1), lambda s, b: (b, s, 0, 0)),
            pl.BlockSpec((1, 1, BS, 1), lambda s, b: (b, s, 0, 0)),
            pl.BlockSpec((1, 1, BS, 1), lambda s, b: (b, s, 0, 0)),
            pl.BlockSpec((16, H), lambda s, b: (0, 0)),
            pl.BlockSpec((1, H), lambda s, b: (0, 0)),
            pl.BlockSpec((1, H), lambda s, b: (0, 0)),
        ],
        out_specs=pl.BlockSpec((1, BS, H), lambda s, b: (b, s, 0)),
        out_shape=jax.ShapeDtypeStruct((B, S, H), jnp.float32),
    )(hidden_states, pos_emb, sin4, cos4, day4, wcat, gamma2, beta2)


def kernel(hidden_states, time_of_day, day_of_week, pos_emb, tod_W, tod_b,
           dow_emb, ln_gamma, ln_beta):
    B, S, H = hidden_states.shape

    wcat = jnp.concatenate(
        [tod_W, dow_emb, tod_b.reshape(1, H),
         jnp.zeros((16 - 3 - dow_emb.shape[0], H), jnp.float32)], axis=0)
    gamma2 = ln_gamma.reshape(1, H)
    beta2 = ln_beta.reshape(1, H)

    devs = jax.devices()
    ndev = len(devs) if B % max(len(devs), 1) == 0 else 1
    if ndev > 1:
        mesh = Mesh(np.array(devs[:ndev]), ("d",))
        fn = shard_map(
            _local_encode, mesh=mesh,
            in_specs=(P("d"), P("d"), P("d"), P(), P(), P(), P()),
            out_specs=P("d"), check_rep=False)
        return fn(hidden_states, time_of_day, day_of_week, pos_emb, wcat,
                  gamma2, beta2)
    return _local_encode(hidden_states, time_of_day, day_of_week, pos_emb,
                         wcat, gamma2, beta2)


# speedup vs baseline: 6.1700x; 6.1700x over previous
import math

import jax
import jax.numpy as jnp
from jax.experimental import pallas as pl

_EPS = 1e-12
_TWO_PI = 2.0 * math.pi


def _sincos_kernel(tod_ref, sin_ref, cos_ref):
    rad = tod_ref[...] * _TWO_PI
    sin_ref[...] = jnp.sin(rad)
    cos_ref[...] = jnp.cos(rad)


def _fused_kernel(hid_ref, pos_ref, sin_ref, cos_ref, day_ref, wcat_ref,
                  gamma_ref, beta_ref, out_ref):
    x = hid_ref[0]
    p = pos_ref[...]
    sin_t = sin_ref[0, 0]
    cos_t = cos_ref[0, 0]
    day = day_ref[0, 0]

    bs = x.shape[0]
    col = jax.lax.broadcasted_iota(jnp.int32, (bs, 16), 1)
    hot = ((col == day + 2) | (col == 9)).astype(jnp.float32)
    m = jnp.where(col == 0, sin_t, jnp.where(col == 1, cos_t, hot))
    extra = jnp.dot(m, wcat_ref[...],
                    preferred_element_type=jnp.float32)

    h = x + p + extra
    inv_h = 1.0 / h.shape[1]
    mean = jnp.sum(h, axis=1, keepdims=True) * inv_h
    msq = jnp.sum(h * h, axis=1, keepdims=True) * inv_h
    var = msq - mean * mean
    rs = jax.lax.rsqrt(var + _EPS)
    out_ref[0] = (h - mean) * rs * gamma_ref[...] + beta_ref[...]


def kernel(hidden_states, time_of_day, day_of_week, pos_emb, tod_W, tod_b,
           dow_emb, ln_gamma, ln_beta):
    B, S, H = hidden_states.shape
    BS = 1024
    NSB = S // BS

    sin_bs, cos_bs = pl.pallas_call(
        _sincos_kernel,
        out_shape=(jax.ShapeDtypeStruct((B, S), jnp.float32),
                   jax.ShapeDtypeStruct((B, S), jnp.float32)),
    )(time_of_day)

    wcat = jnp.concatenate(
        [tod_W, dow_emb, tod_b.reshape(1, H),
         jnp.zeros((16 - 3 - dow_emb.shape[0], H), jnp.float32)], axis=0)

    sin4 = sin_bs.reshape(B, NSB, BS, 1)
    cos4 = cos_bs.reshape(B, NSB, BS, 1)
    day4 = day_of_week.astype(jnp.int32).reshape(B, NSB, BS, 1)
    gamma2 = ln_gamma.reshape(1, H)
    beta2 = ln_beta.reshape(1, H)

    grid = (NSB, B)
    out = pl.pallas_call(
        _fused_kernel,
        grid=grid,
        in_specs=[
            pl.BlockSpec((1, BS, H), lambda s, b: (b, s, 0)),
            pl.BlockSpec((BS, H), lambda s, b: (s, 0)),
            pl.BlockSpec((1, 1, BS, 1), lambda s, b: (b, s, 0, 0)),
            pl.BlockSpec((1, 1, BS, 1), lambda s, b: (b, s, 0, 0)),
            pl.BlockSpec((1, 1, BS, 1), lambda s, b: (b, s, 0, 0)),
            pl.BlockSpec((16, H), lambda s, b: (0, 0)),
            pl.BlockSpec((1, H), lambda s, b: (0, 0)),
            pl.BlockSpec((1, H), lambda s, b: (0, 0)),
        ],
        out_specs=pl.BlockSpec((1, BS, H), lambda s, b: (b, s, 0)),
        out_shape=jax.ShapeDtypeStruct((B, S, H), jnp.float32),
    )(hidden_states, pos_emb, sin4, cos4, day4, wcat, gamma2, beta2)
    return out
